# Initial kernel scaffold; baseline (speedup 1.0000x reference)
import jax
import jax.numpy as jnp
from jax.experimental import pallas as pl


def kernel(feat, edge_index, params):
    raise NotImplementedError("write your pallas kernel here")


# speedup vs baseline: 20.0791x; 20.0791x over previous
import functools

import jax
import jax.numpy as jnp
from jax import lax
from jax.experimental import pallas as pl
from jax.experimental.pallas import tpu as pltpu
from jax.experimental.pallas import tpu_sc as plsc

N = 10000
F = 128
NPAD = 10112
E2 = 320000 + N
TILES = 32
CW = 64
CH = 164
CHA = 166
EPT = CH * CW
E2P = TILES * EPT
RPT = NPAD // 16


def _lane_bcast(v, i):
    return lax.gather(
        v,
        jnp.full((16, 1), i, jnp.int32),
        lax.GatherDimensionNumbers(
            offset_dims=(), collapsed_slice_dims=(0,), start_index_map=(0,)),
        (1,),
        mode=lax.GatherScatterMode.PROMISE_IN_BOUNDS)


def _emit_gat_head(x, w_ref, atts_ref, attd_ref, h_ref, as_ref, ad_ref, m_ref):
    h = jnp.dot(x, w_ref[...], preferred_element_type=jnp.float32)
    h_ref[...] = h
    a_s = jnp.sum(h * atts_ref[...], axis=1, keepdims=True)
    a_d = jnp.sum(h * attd_ref[...], axis=1, keepdims=True)
    as_ref[...] = a_s
    ad_ref[...] = a_d
    m_ref[...] = jnp.full((8, 128), jnp.max(a_s), jnp.float32)


def _first_body(x_ref, w_ref, atts_ref, attd_ref, h_ref, as_ref, ad_ref, m_ref):
    _emit_gat_head(x_ref[...], w_ref, atts_ref, attd_ref,
                   h_ref, as_ref, ad_ref, m_ref)


def _stage_body(agg_ref, den_ref, bias_ref, gam_ref, bet_ref, mu_ref, var_ref,
                w_ref, atts_ref, attd_ref, h_ref, as_ref, ad_ref, m_ref):
    a = agg_ref[0] + agg_ref[1]
    dn = jnp.sum(den_ref[...], axis=0)[:, None] + 1e-16
    x = jnp.maximum(a / dn + bias_ref[...], 0.0)
    x = (x - mu_ref[...]) * (gam_ref[...] * lax.rsqrt(var_ref[...] + 1e-5)) \
        + bet_ref[...]
    rows = lax.broadcasted_iota(jnp.int32, (NPAD, 1), 0)
    x = jnp.where(rows < N, x, 0.0)
    _emit_gat_head(x, w_ref, atts_ref, attd_ref, h_ref, as_ref, ad_ref, m_ref)


def _final_body(agg_ref, den_ref, bias_ref, out_ref):
    a = agg_ref[0] + agg_ref[1]
    dn = jnp.sum(den_ref[...], axis=0)[:, None] + 1e-16
    out_ref[...] = jnp.maximum(a / dn + bias_ref[...], 0.0)


_HEAD_OUT = [
    jax.ShapeDtypeStruct((NPAD, F), jnp.float32),
    jax.ShapeDtypeStruct((NPAD, 1), jnp.float32),
    jax.ShapeDtypeStruct((NPAD, 1), jnp.float32),
    jax.ShapeDtypeStruct((8, 128), jnp.float32),
]

_tc_first = pl.pallas_call(_first_body, out_shape=_HEAD_OUT)
_tc_stage = pl.pallas_call(_stage_body, out_shape=_HEAD_OUT)
_tc_final = pl.pallas_call(
    _final_body, out_shape=jax.ShapeDtypeStruct((NPAD, F), jnp.float32))


_mesh = plsc.VectorSubcoreMesh(core_axis_name="c", subcore_axis_name="s")


@functools.partial(
    pl.kernel,
    out_type=[
        jax.ShapeDtypeStruct((2, NPAD, F), jnp.float32),
        jax.ShapeDtypeStruct((TILES * NPAD,), jnp.float32),
    ],
    mesh=_mesh,
    compiler_params=pltpu.CompilerParams(needs_layout_passes=False),
    scratch_types=[
        pltpu.VMEM((CHA * CW,), jnp.int32),
        pltpu.VMEM((CHA * CW,), jnp.int32),
        pltpu.VMEM((8, CW), jnp.int32),
        pltpu.VMEM((16,), jnp.float32),
        pltpu.VMEM((NPAD,), jnp.float32),
        pltpu.VMEM((CW,), jnp.float32),
        pltpu.VMEM((CW,), jnp.float32),
        pltpu.VMEM((CW,), jnp.float32),
        pltpu.VMEM((CW,), jnp.float32),
        pltpu.VMEM((CW,), jnp.float32),
        pltpu.VMEM((CW, F), jnp.float32),
        pltpu.VMEM((CW, F), jnp.float32),
        pltpu.VMEM_SHARED((NPAD, F), jnp.float32),
        pltpu.SemaphoreType.DMA,
        pltpu.SemaphoreType.DMA,
    ],
)
def _sc_edge(src_hbm, dst_hbm, as_hbm, ad_hbm, m_hbm, h_hbm,
             agg_out, den_out,
             srcv, dstv, dsts, mv, denv, gaA, gaB, gbA, gbB, eb, rba, rbb,
             aggsh, sema, semb):
    cid = lax.axis_index("c")
    sid = lax.axis_index("s")
    wid = cid * 16 + sid

    pltpu.sync_copy(src_hbm.at[pl.ds(wid * CHA * CW, CHA * CW)], srcv)
    pltpu.sync_copy(dst_hbm.at[pl.ds(wid * CHA * CW, CHA * CW)], dstv)
    pltpu.sync_copy(m_hbm, mv)

    zero16 = jnp.zeros((16,), jnp.float32)

    def _zd(i, c):
        denv[pl.ds(i * 16, 16)] = zero16
        return c

    lax.fori_loop(0, NPAD // 16, _zd, 0)

    def _zr(r, c):
        for v in range(F // 16):
            rba[r, pl.ds(v * 16, 16)] = zero16
        return c

    lax.fori_loop(0, CW, _zr, 0)

    base = sid * RPT
    for k in range(9):
        pltpu.sync_copy(rba, aggsh.at[pl.ds(base + k * CW, CW)])
    pltpu.sync_copy(rba.at[pl.ds(0, RPT - 9 * CW)],
                    aggsh.at[pl.ds(base + 9 * CW, RPT - 9 * CW)])
    plsc.subcore_barrier()

    Mv = mv[...]

    def _fetch(j, ga, gb, buf, sem):
        pltpu.async_copy(h_hbm.at[srcv.at[pl.ds(j * CW, CW)]], buf, sem)
        pltpu.async_copy(as_hbm.at[srcv.at[pl.ds(j * CW, CW)]], ga, sem)
        pltpu.async_copy(ad_hbm.at[dstv.at[pl.ds(j * CW, CW)]], gb, sem)

    def _wait(j, ga, gb, buf, sem):
        pltpu.make_async_copy(h_hbm.at[srcv.at[pl.ds(j * CW, CW)]],
                              buf, sem).wait()
        pltpu.make_async_copy(as_hbm.at[srcv.at[pl.ds(j * CW, CW)]],
                              ga, sem).wait()
        pltpu.make_async_copy(ad_hbm.at[dstv.at[pl.ds(j * CW, CW)]],
                              gb, sem).wait()

    def _process(j, row, ga, gb, buf):
        for g in range(CW // 16):
            sl = pl.ds(g * 16, 16)
            va = ga[sl]
            vb = gb[sl]
            d = dstv[pl.ds(j * CW + g * 16, 16)]
            dsts[row, sl] = d
            t = va + vb
            t = jnp.maximum(t, 0.2 * t)
            c = vb + Mv
            c = jnp.maximum(c, 0.2 * c)
            e = jnp.exp(t - c)
            eb[sl] = e
            plsc.addupdate_scatter(denv, [d], e)

        def _grp(g, c):
            evec = eb[pl.ds(g * 16, 16)]
            for i in range(16):
                r = g * 16 + i
                ebc = _lane_bcast(evec, i)
                for v in range(F // 16):
                    sl = pl.ds(v * 16, 16)
                    buf[r, sl] = buf[r, sl] * ebc
            return c

        lax.fori_loop(0, CW // 16, _grp, 0)
        pltpu.sync_copy(buf, aggsh.at[dsts.at[row]], add=True)

    _fetch(0, gaA, gbA, rba, sema)

    def _pair(p, c):
        j0 = 2 * p
        _fetch(j0 + 1, gaB, gbB, rbb, semb)
        _wait(j0, gaA, gbA, rba, sema)
        _process(j0, 0, gaA, gbA, rba)
        _fetch(j0 + 2, gaA, gbA, rba, sema)
        _wait(j0 + 1, gaB, gbB, rbb, semb)
        _process(j0 + 1, 1, gaB, gbB, rbb)
        return c

    lax.fori_loop(0, CH // 2, _pair, 0)
    _wait(CH, gaA, gbA, rba, sema)

    plsc.subcore_barrier()
    for k in range(9):
        pltpu.sync_copy(aggsh.at[pl.ds(base + k * CW, CW)],
                        agg_out.at[cid, pl.ds(base + k * CW, CW)])
    rem = RPT - 9 * CW
    pltpu.sync_copy(aggsh.at[pl.ds(base + 9 * CW, rem)],
                    agg_out.at[cid, pl.ds(base + 9 * CW, rem)])
    pltpu.sync_copy(denv, den_out.at[pl.ds(wid * NPAD, NPAD)])


def _layer_inputs(p):
    return (p["W"], p["att_src"].reshape(1, F), p["att_dst"].reshape(1, F))


def kernel(feat, edge_index, params):
    x = jnp.pad(feat, ((0, NPAD - N), (0, 0)))
    loop = jnp.arange(N, dtype=edge_index.dtype)
    src = jnp.concatenate([edge_index[0], loop])
    dst = jnp.concatenate([edge_index[1], loop])
    src = jnp.pad(src, (0, E2P - E2))
    dst = jnp.pad(dst, (0, E2P - E2), constant_values=N)
    src3 = jnp.pad(src.reshape(TILES, EPT),
                   ((0, 0), (0, (CHA - CH) * CW))).reshape(-1)
    dst3 = jnp.pad(dst.reshape(TILES, EPT), ((0, 0), (0, (CHA - CH) * CW)),
                   constant_values=N).reshape(-1)

    def edge_phase(h, a_s, a_d, m8):
        agg, den = _sc_edge(src3, dst3, a_s.reshape(NPAD), a_d.reshape(NPAD),
                            m8[0, :16], h)
        return agg, den.reshape(TILES, NPAD)

    p = params
    h, a_s, a_d, m8 = _tc_first(x, *_layer_inputs(p["gc1e"]))
    agg, den = edge_phase(h, a_s, a_d, m8)
    for prev, bn, cur in (("gc1e", "bn1e", "gc2e"),
                          ("gc2e", "bn2e", "gc1d"),
                          ("gc1d", "bn1d", "gc2d")):
        b = p[bn]
        h, a_s, a_d, m8 = _tc_stage(
            agg, den, p[prev]["bias"].reshape(1, F),
            b["gamma"].reshape(1, F), b["beta"].reshape(1, F),
            b["mean"].reshape(1, F), b["var"].reshape(1, F),
            *_layer_inputs(p[cur]))
        agg, den = edge_phase(h, a_s, a_d, m8)
    out = _tc_final(agg, den, p["gc2d"]["bias"].reshape(1, F))
    return out[:N]
